# baseline (speedup 1.0000x reference)
import jax
import jax.numpy as jnp
from jax.experimental import pallas as pl
from jax.experimental.pallas import tpu as pltpu

_R = 256
_K = 16


def kernel(x, pos_emb):
    B, T, C = x.shape
    x2 = x.reshape(B * T, C)
    N = (B * T) // _R
    P = T // _R

    def body(x_hbm, pe_hbm, o_hbm, xbuf, pebuf, obuf, rsem, psem, wsem):
        def mk_read(i):
            return pltpu.make_async_copy(
                x_hbm.at[pl.ds(i * _R, _R), :], xbuf.at[i % _K], rsem.at[i % _K]
            )

        def mk_write(i):
            return pltpu.make_async_copy(
                obuf.at[i % _K], o_hbm.at[pl.ds(i * _R, _R), :], wsem.at[i % _K]
            )

        pe_reads = []
        for j in range(P):
            c = pltpu.make_async_copy(
                pe_hbm.at[pl.ds(j * _R, _R), :], pebuf.at[j], psem.at[j]
            )
            c.start()
            pe_reads.append(c)

        reads = {}
        writes = {}
        for i in range(_K):
            reads[i] = mk_read(i)
            reads[i].start()

        for i in range(N):
            reads[i].wait()
            if i < P:
                pe_reads[i].wait()
            if i >= _K:
                writes[i - _K].wait()
            obuf[i % _K, :, :] = xbuf[i % _K, :, :] + pebuf[i % P, :, :]
            writes[i] = mk_write(i)
            writes[i].start()
            if i + _K < N:
                reads[i + _K] = mk_read(i + _K)
                reads[i + _K].start()

        for i in range(N - _K, N):
            writes[i].wait()

    out = pl.pallas_call(
        body,
        in_specs=[
            pl.BlockSpec(memory_space=pltpu.MemorySpace.HBM),
            pl.BlockSpec(memory_space=pltpu.MemorySpace.HBM),
        ],
        out_specs=pl.BlockSpec(memory_space=pltpu.MemorySpace.HBM),
        out_shape=jax.ShapeDtypeStruct((B * T, C), x.dtype),
        scratch_shapes=[
            pltpu.VMEM((_K, _R, C), x.dtype),
            pltpu.VMEM((P, _R, C), x.dtype),
            pltpu.VMEM((_K, _R, C), x.dtype),
            pltpu.SemaphoreType.DMA((_K,)),
            pltpu.SemaphoreType.DMA((P,)),
            pltpu.SemaphoreType.DMA((_K,)),
        ],
    )(x2, pos_emb)
    return out.reshape(B, T, C)


# speedup vs baseline: 1.0277x; 1.0277x over previous
import jax
import jax.numpy as jnp
from jax.experimental import pallas as pl
from jax.experimental.pallas import tpu as pltpu

_R = 512
_K = 12


def kernel(x, pos_emb):
    B, T, C = x.shape
    x2 = x.reshape(B * T, C)
    N = (B * T) // _R
    P = T // _R

    def body(x_hbm, pe_hbm, o_hbm, xbuf, pebuf, obuf, rsem, psem, wsem):
        def mk_read(i):
            return pltpu.make_async_copy(
                x_hbm.at[pl.ds(i * _R, _R), :], xbuf.at[i % _K], rsem.at[i % _K]
            )

        def mk_write(i):
            return pltpu.make_async_copy(
                obuf.at[i % _K], o_hbm.at[pl.ds(i * _R, _R), :], wsem.at[i % _K]
            )

        pe_reads = []
        for j in range(P):
            c = pltpu.make_async_copy(
                pe_hbm.at[pl.ds(j * _R, _R), :], pebuf.at[j], psem.at[j]
            )
            c.start()
            pe_reads.append(c)

        reads = {}
        writes = {}
        for i in range(_K):
            reads[i] = mk_read(i)
            reads[i].start()

        for i in range(N):
            reads[i].wait()
            if i < P:
                pe_reads[i].wait()
            if i >= _K:
                writes[i - _K].wait()
            obuf[i % _K, :, :] = xbuf[i % _K, :, :] + pebuf[i % P, :, :]
            writes[i] = mk_write(i)
            writes[i].start()
            if i + _K < N:
                reads[i + _K] = mk_read(i + _K)
                reads[i + _K].start()

        for i in range(N - _K, N):
            writes[i].wait()

    out = pl.pallas_call(
        body,
        in_specs=[
            pl.BlockSpec(memory_space=pltpu.MemorySpace.HBM),
            pl.BlockSpec(memory_space=pltpu.MemorySpace.HBM),
        ],
        out_specs=pl.BlockSpec(memory_space=pltpu.MemorySpace.HBM),
        out_shape=jax.ShapeDtypeStruct((B * T, C), x.dtype),
        scratch_shapes=[
            pltpu.VMEM((_K, _R, C), x.dtype),
            pltpu.VMEM((P, _R, C), x.dtype),
            pltpu.VMEM((_K, _R, C), x.dtype),
            pltpu.SemaphoreType.DMA((_K,)),
            pltpu.SemaphoreType.DMA((P,)),
            pltpu.SemaphoreType.DMA((_K,)),
        ],
    )(x2, pos_emb)
    return out.reshape(B, T, C)
